# initial kernel scaffold (unmeasured)
import jax
import jax.numpy as jnp
from jax import lax
from jax.experimental import pallas as pl
from jax.experimental.pallas import tpu as pltpu

N_DEV = 16


def kernel(x, w_mat):
    m_per, k = x.shape
    _, n = w_mat.shape

    def body(x_ref, w_ref, out_ref, comm_ref, send_sems, recv_sems, ack_sem,
             copy_sem):
        my = lax.axis_index("i")
        left = (my - 1 + N_DEV) % N_DEV
        right = (my + 1) % N_DEV

        barrier = pltpu.get_barrier_semaphore()
        for nbr in (left, right):
            pl.semaphore_signal(barrier, inc=1, device_id=(nbr,),
                                device_id_type=pl.DeviceIdType.MESH)
        pl.semaphore_wait(barrier, 2)

        cp = pltpu.make_async_copy(x_ref, comm_ref.at[0], copy_sem)
        cp.start()
        cp.wait()
        out_ref[pl.ds(my * m_per, m_per), :] = jnp.dot(
            comm_ref[0], w_ref[:], preferred_element_type=jnp.float32)

        for h in range(N_DEV - 1):
            send_slot = h % 2
            recv_slot = (h + 1) % 2
            if h >= 1:
                pl.semaphore_wait(ack_sem, 1)
            rdma = pltpu.make_async_remote_copy(
                src_ref=comm_ref.at[send_slot],
                dst_ref=comm_ref.at[recv_slot],
                send_sem=send_sems.at[send_slot],
                recv_sem=recv_sems.at[recv_slot],
                device_id=(right,),
                device_id_type=pl.DeviceIdType.MESH,
            )
            rdma.start()
            rdma.wait()
            pl.semaphore_signal(ack_sem, inc=1, device_id=(left,),
                                device_id_type=pl.DeviceIdType.MESH)
            origin = (my - h - 1 + N_DEV) % N_DEV
            out_ref[pl.ds(origin * m_per, m_per), :] = jnp.dot(
                comm_ref[recv_slot], w_ref[:],
                preferred_element_type=jnp.float32)

    return pl.pallas_call(
        body,
        out_shape=jax.ShapeDtypeStruct((N_DEV * m_per, n), jnp.float32),
        in_specs=[
            pl.BlockSpec(memory_space=pltpu.ANY),
            pl.BlockSpec(memory_space=pltpu.VMEM),
        ],
        out_specs=pl.BlockSpec(memory_space=pltpu.VMEM),
        scratch_shapes=[
            pltpu.VMEM((2, m_per, k), jnp.float32),
            pltpu.SemaphoreType.DMA((2,)),
            pltpu.SemaphoreType.DMA((2,)),
            pltpu.SemaphoreType.REGULAR,
            pltpu.SemaphoreType.DMA,
        ],
        compiler_params=pltpu.CompilerParams(collective_id=0),
    )(x, w_mat)


# baseline (device time: 2816675 ns/iter reference)
import jax
import jax.numpy as jnp
from jax import lax
from jax.experimental import pallas as pl
from jax.experimental.pallas import tpu as pltpu

N_DEV = 16


def kernel(x, w_mat):
    m_per, k = x.shape
    _, n = w_mat.shape

    def body(x_ref, w_ref, out_ref, comm_ref, send_sems, recv_sems, ack_sem,
             copy_sem):
        my = lax.axis_index("i")
        left = (my - 1 + N_DEV) % N_DEV
        right = (my + 1) % N_DEV

        barrier = pltpu.get_barrier_semaphore()
        for nbr in (left, right):
            pl.semaphore_signal(barrier, inc=1, device_id=(nbr,),
                                device_id_type=pl.DeviceIdType.MESH)
        pl.semaphore_wait(barrier, 2)

        cp = pltpu.make_async_copy(x_ref, comm_ref.at[0], copy_sem)
        cp.start()
        cp.wait()
        out_ref[pl.ds(my * m_per, m_per), :] = jnp.dot(
            comm_ref[0], w_ref[:], preferred_element_type=jnp.float32)

        for h in range(N_DEV - 1):
            send_slot = h % 2
            recv_slot = (h + 1) % 2
            rdma = pltpu.make_async_remote_copy(
                src_ref=comm_ref.at[send_slot],
                dst_ref=comm_ref.at[recv_slot],
                send_sem=send_sems.at[send_slot],
                recv_sem=recv_sems.at[recv_slot],
                device_id=(right,),
                device_id_type=pl.DeviceIdType.MESH,
            )
            rdma.start()
            rdma.wait()
            origin = (my - h - 1 + N_DEV) % N_DEV
            out_ref[pl.ds(origin * m_per, m_per), :] = jnp.dot(
                comm_ref[recv_slot], w_ref[:],
                preferred_element_type=jnp.float32)

    return pl.pallas_call(
        body,
        out_shape=jax.ShapeDtypeStruct((N_DEV * m_per, n), jnp.float32),
        in_specs=[
            pl.BlockSpec(memory_space=pltpu.HBM),
            pl.BlockSpec(memory_space=pltpu.VMEM),
        ],
        out_specs=pl.BlockSpec(memory_space=pltpu.VMEM),
        scratch_shapes=[
            pltpu.VMEM((2, m_per, k), jnp.float32),
            pltpu.SemaphoreType.DMA((2,)),
            pltpu.SemaphoreType.DMA((2,)),
            pltpu.SemaphoreType.REGULAR,
            pltpu.SemaphoreType.DMA,
        ],
        compiler_params=pltpu.CompilerParams(collective_id=0),
    )(x, w_mat)


# device time: 1420938 ns/iter; 1.9823x vs baseline; 1.9823x over previous
import jax
import jax.numpy as jnp
from jax import lax
from jax.experimental import pallas as pl
from jax.experimental.pallas import tpu as pltpu

N_DEV = 16


def kernel(x, w_mat):
    m_per, k = x.shape
    _, n = w_mat.shape
    m_half = m_per // 2

    def body(x_ref, w_ref, out_ref, cw_comm, ccw_comm,
             cw_send_sems, cw_recv_sems, ccw_send_sems, ccw_recv_sems,
             copy_sems):
        my = lax.axis_index("i")
        left = (my - 1 + N_DEV) % N_DEV
        right = (my + 1) % N_DEV

        barrier = pltpu.get_barrier_semaphore()
        for nbr in (left, right):
            pl.semaphore_signal(barrier, inc=1, device_id=(nbr,),
                                device_id_type=pl.DeviceIdType.MESH)
        pl.semaphore_wait(barrier, 2)

        cp_top = pltpu.make_async_copy(
            x_ref.at[pl.ds(0, m_half)], cw_comm.at[0], copy_sems.at[0])
        cp_bot = pltpu.make_async_copy(
            x_ref.at[pl.ds(m_half, m_half)], ccw_comm.at[0], copy_sems.at[1])
        cp_top.start()
        cp_bot.start()
        cp_top.wait()
        cp_bot.wait()

        for t in range(N_DEV):
            s = t % 2
            if t < N_DEV - 1:
                cw = pltpu.make_async_remote_copy(
                    src_ref=cw_comm.at[s],
                    dst_ref=cw_comm.at[1 - s],
                    send_sem=cw_send_sems.at[s],
                    recv_sem=cw_recv_sems.at[1 - s],
                    device_id=(right,),
                    device_id_type=pl.DeviceIdType.MESH,
                )
                ccw = pltpu.make_async_remote_copy(
                    src_ref=ccw_comm.at[s],
                    dst_ref=ccw_comm.at[1 - s],
                    send_sem=ccw_send_sems.at[s],
                    recv_sem=ccw_recv_sems.at[1 - s],
                    device_id=(left,),
                    device_id_type=pl.DeviceIdType.MESH,
                )
                cw.start()
                ccw.start()
            o_cw = (my - t + N_DEV) % N_DEV
            o_ccw = (my + t) % N_DEV
            out_ref[pl.ds(o_cw * m_per, m_half), :] = jnp.dot(
                cw_comm[s], w_ref[:], preferred_element_type=jnp.float32)
            out_ref[pl.ds(o_ccw * m_per + m_half, m_half), :] = jnp.dot(
                ccw_comm[s], w_ref[:], preferred_element_type=jnp.float32)
            if t < N_DEV - 1:
                cw.wait()
                ccw.wait()

    return pl.pallas_call(
        body,
        out_shape=jax.ShapeDtypeStruct((N_DEV * m_per, n), jnp.float32),
        in_specs=[
            pl.BlockSpec(memory_space=pltpu.HBM),
            pl.BlockSpec(memory_space=pltpu.VMEM),
        ],
        out_specs=pl.BlockSpec(memory_space=pltpu.VMEM),
        scratch_shapes=[
            pltpu.VMEM((2, m_half, k), jnp.float32),
            pltpu.VMEM((2, m_half, k), jnp.float32),
            pltpu.SemaphoreType.DMA((2,)),
            pltpu.SemaphoreType.DMA((2,)),
            pltpu.SemaphoreType.DMA((2,)),
            pltpu.SemaphoreType.DMA((2,)),
            pltpu.SemaphoreType.DMA((2,)),
        ],
        compiler_params=pltpu.CompilerParams(
            collective_id=0,
            vmem_limit_bytes=56 * 1024 * 1024,
        ),
    )(x, w_mat)
